# baseline (speedup 1.0000x reference)
import functools

import jax
import jax.numpy as jnp
from jax import lax
from jax.experimental import pallas as pl
from jax.experimental.pallas import tpu as pltpu
from jax.experimental.pallas import tpu_sc as plsc

V = 100000
E = 1024
H = 1024
B = 128

_BV = 2048
_NV = (V + _BV - 1) // _BV

_NC = 2
_GW = 16
_RPW = B // _GW


def _sc_gather(ids, emb):
    mesh = plsc.VectorSubcoreMesh(core_axis_name="c", subcore_axis_name="s")

    @functools.partial(
        pl.kernel,
        mesh=mesh,
        out_type=jax.ShapeDtypeStruct((B, E), jnp.float32),
        scratch_types=[
            pltpu.VMEM((_RPW,), jnp.int32),
            pltpu.VMEM((_RPW, E), jnp.float32),
            pltpu.SemaphoreType.DMA,
        ],
    )
    def gather_kernel(ids_hbm, emb_hbm, x_hbm, idx_v, rows_v, sem):
        wid = lax.axis_index("s") * _NC + lax.axis_index("c")

        @pl.when(wid < _GW)
        def _():
            base = wid * _RPW
            pltpu.sync_copy(ids_hbm.at[pl.ds(base, _RPW)], idx_v)
            pltpu.async_copy(emb_hbm.at[idx_v], rows_v, sem).wait()
            pltpu.sync_copy(rows_v, x_hbm.at[pl.ds(base, _RPW)])

    return gather_kernel(ids, emb)


def _lstm_fc_body(x_ref, h_ref, c_ref, wih_ref, whh_ref, bih_ref, bhh_ref,
                  fcw_ref, fcb_ref, pred_ref, hout_ref, cout_ref):
    nt = (((1,), (1,)), ((), ()))

    @pl.when(pl.program_id(0) == 0)
    def _():
        gates = (
            lax.dot_general(x_ref[...], wih_ref[...], nt,
                            preferred_element_type=jnp.float32)
            + lax.dot_general(h_ref[...], whh_ref[...], nt,
                              preferred_element_type=jnp.float32)
            + bih_ref[...] + bhh_ref[...]
        )
        i_g = jax.nn.sigmoid(gates[:, 0:H])
        f_g = jax.nn.sigmoid(gates[:, H:2 * H])
        g_g = jnp.tanh(gates[:, 2 * H:3 * H])
        o_g = jax.nn.sigmoid(gates[:, 3 * H:4 * H])
        c_new = f_g * c_ref[...] + i_g * g_g
        cout_ref[...] = c_new
        hout_ref[...] = o_g * jnp.tanh(c_new)

    pred_ref[...] = (
        lax.dot_general(hout_ref[...], fcw_ref[...], nt,
                        preferred_element_type=jnp.float32)
        + fcb_ref[...]
    )


def _lstm_fc(x, h, c, W_ih, W_hh, b_ih2, b_hh2, fc_W, fc_b2):
    return pl.pallas_call(
        _lstm_fc_body,
        grid=(_NV,),
        in_specs=[
            pl.BlockSpec((B, E), lambda i: (0, 0)),
            pl.BlockSpec((B, H), lambda i: (0, 0)),
            pl.BlockSpec((B, H), lambda i: (0, 0)),
            pl.BlockSpec((4 * H, E), lambda i: (0, 0)),
            pl.BlockSpec((4 * H, H), lambda i: (0, 0)),
            pl.BlockSpec((1, 4 * H), lambda i: (0, 0)),
            pl.BlockSpec((1, 4 * H), lambda i: (0, 0)),
            pl.BlockSpec((_BV, H), lambda i: (i, 0)),
            pl.BlockSpec((1, _BV), lambda i: (0, i)),
        ],
        out_specs=[
            pl.BlockSpec((B, _BV), lambda i: (0, i)),
            pl.BlockSpec((B, H), lambda i: (0, 0)),
            pl.BlockSpec((B, H), lambda i: (0, 0)),
        ],
        out_shape=[
            jax.ShapeDtypeStruct((B, V), jnp.float32),
            jax.ShapeDtypeStruct((B, H), jnp.float32),
            jax.ShapeDtypeStruct((B, H), jnp.float32),
        ],
    )(x, h, c, W_ih, W_hh, b_ih2, b_hh2, fc_W, fc_b2)


def kernel(input, h0, c0, emb, W_ih, W_hh, b_ih, b_hh, fc_W, fc_b):
    ids = input.astype(jnp.int32)
    x = _sc_gather(ids, emb)
    pred, h_new, c_new = _lstm_fc(
        x, h0[0], c0[0], W_ih, W_hh,
        b_ih.reshape(1, 4 * H), b_hh.reshape(1, 4 * H),
        fc_W, fc_b.reshape(1, V),
    )
    return (pred, h_new[None, :, :], c_new[None, :, :])


# speedup vs baseline: 1.0009x; 1.0009x over previous
import functools

import jax
import jax.numpy as jnp
from jax import lax
from jax.experimental import pallas as pl
from jax.experimental.pallas import tpu as pltpu
from jax.experimental.pallas import tpu_sc as plsc

V = 100000
E = 1024
H = 1024
B = 128

_KS = 2
_SUB = 1024
_BV = _KS * _SUB
_NV = (V + _BV - 1) // _BV

_NC = 2
_GW = 16
_RPW = B // _GW


def _sc_gather(ids, emb):
    mesh = plsc.VectorSubcoreMesh(core_axis_name="c", subcore_axis_name="s")

    @functools.partial(
        pl.kernel,
        mesh=mesh,
        out_type=jax.ShapeDtypeStruct((B, E), jnp.float32),
        scratch_types=[
            pltpu.VMEM((_RPW,), jnp.int32),
            pltpu.VMEM((_RPW, E), jnp.float32),
            pltpu.SemaphoreType.DMA,
        ],
    )
    def gather_kernel(ids_hbm, emb_hbm, x_hbm, idx_v, rows_v, sem):
        wid = lax.axis_index("s") * _NC + lax.axis_index("c")

        @pl.when(wid < _GW)
        def _():
            base = wid * _RPW
            pltpu.sync_copy(ids_hbm.at[pl.ds(base, _RPW)], idx_v)
            pltpu.async_copy(emb_hbm.at[idx_v], rows_v, sem).wait()
            pltpu.sync_copy(rows_v, x_hbm.at[pl.ds(base, _RPW)])

    return gather_kernel(ids, emb)


def _lstm_fc_body(x_ref, h_ref, c_ref, wih_ref, whh_ref, bih_ref, bhh_ref,
                  *rest):
    fcw_refs = rest[:_KS]
    fcb_ref, pred_ref, hout_ref, cout_ref = rest[_KS:]
    nt = (((1,), (1,)), ((), ()))

    @pl.when(pl.program_id(0) == 0)
    def _():
        gates = (
            lax.dot_general(x_ref[...], wih_ref[...], nt,
                            preferred_element_type=jnp.float32)
            + lax.dot_general(h_ref[...], whh_ref[...], nt,
                              preferred_element_type=jnp.float32)
            + bih_ref[...] + bhh_ref[...]
        )
        i_g = jax.nn.sigmoid(gates[:, 0:H])
        f_g = jax.nn.sigmoid(gates[:, H:2 * H])
        g_g = jnp.tanh(gates[:, 2 * H:3 * H])
        o_g = jax.nn.sigmoid(gates[:, 3 * H:4 * H])
        c_new = f_g * c_ref[...] + i_g * g_g
        cout_ref[...] = c_new
        hout_ref[...] = o_g * jnp.tanh(c_new)

    h_new = hout_ref[...]
    parts = [
        lax.dot_general(h_new, fcw_refs[j][...], nt,
                        preferred_element_type=jnp.float32)
        for j in range(_KS)
    ]
    pred_ref[...] = jnp.concatenate(parts, axis=1) + fcb_ref[...]


def _lstm_fc(x, h, c, W_ih, W_hh, b_ih2, b_hh2, fc_W, fc_b2):
    return pl.pallas_call(
        _lstm_fc_body,
        grid=(_NV,),
        in_specs=[
            pl.BlockSpec((B, E), lambda i: (0, 0)),
            pl.BlockSpec((B, H), lambda i: (0, 0)),
            pl.BlockSpec((B, H), lambda i: (0, 0)),
            pl.BlockSpec((4 * H, E), lambda i: (0, 0)),
            pl.BlockSpec((4 * H, H), lambda i: (0, 0)),
            pl.BlockSpec((1, 4 * H), lambda i: (0, 0)),
            pl.BlockSpec((1, 4 * H), lambda i: (0, 0)),
        ] + [
            pl.BlockSpec((_SUB, H), lambda i, j=j: (_KS * i + j, 0))
            for j in range(_KS)
        ] + [
            pl.BlockSpec((1, _BV), lambda i: (0, i)),
        ],
        out_specs=[
            pl.BlockSpec((B, _BV), lambda i: (0, i)),
            pl.BlockSpec((B, H), lambda i: (0, 0)),
            pl.BlockSpec((B, H), lambda i: (0, 0)),
        ],
        out_shape=[
            jax.ShapeDtypeStruct((B, V), jnp.float32),
            jax.ShapeDtypeStruct((B, H), jnp.float32),
            jax.ShapeDtypeStruct((B, H), jnp.float32),
        ],
    )(x, h, c, W_ih, W_hh, b_ih2, b_hh2, *([fc_W] * _KS), fc_b2)


def kernel(input, h0, c0, emb, W_ih, W_hh, b_ih, b_hh, fc_W, fc_b):
    ids = input.astype(jnp.int32)
    x = _sc_gather(ids, emb)
    pred, h_new, c_new = _lstm_fc(
        x, h0[0], c0[0], W_ih, W_hh,
        b_ih.reshape(1, 4 * H), b_hh.reshape(1, 4 * H),
        fc_W, fc_b.reshape(1, V),
    )
    return (pred, h_new[None, :, :], c_new[None, :, :])


# speedup vs baseline: 1.0058x; 1.0049x over previous
import functools

import jax
import jax.numpy as jnp
from jax import lax
from jax.experimental import pallas as pl
from jax.experimental.pallas import tpu as pltpu
from jax.experimental.pallas import tpu_sc as plsc

V = 100000
E = 1024
H = 1024
B = 128

_SUB = 2048
_NT = V // _SUB
_TAIL = V - _NT * _SUB
_NV = _NT + 1
_NBUF = 4

_NC = 2
_GW = 16
_RPW = B // _GW

_nt_dims = (((1,), (1,)), ((), ()))


def _sc_gather(ids, emb):
    mesh = plsc.VectorSubcoreMesh(core_axis_name="c", subcore_axis_name="s")

    @functools.partial(
        pl.kernel,
        mesh=mesh,
        out_type=jax.ShapeDtypeStruct((B, E), jnp.float32),
        scratch_types=[
            pltpu.VMEM((_RPW,), jnp.int32),
            pltpu.VMEM((_RPW, E), jnp.float32),
            pltpu.SemaphoreType.DMA,
        ],
    )
    def gather_kernel(ids_hbm, emb_hbm, x_hbm, idx_v, rows_v, sem):
        wid = lax.axis_index("s") * _NC + lax.axis_index("c")

        @pl.when(wid < _GW)
        def _():
            base = wid * _RPW
            pltpu.sync_copy(ids_hbm.at[pl.ds(base, _RPW)], idx_v)
            pltpu.async_copy(emb_hbm.at[idx_v], rows_v, sem).wait()
            pltpu.sync_copy(rows_v, x_hbm.at[pl.ds(base, _RPW)])

    return gather_kernel(ids, emb)


def _lstm_body(x_ref, h_ref, c_ref, wih_ref, whh_ref, bih_ref, bhh_ref,
               hout_ref, cout_ref):
    gates = (
        lax.dot_general(x_ref[...], wih_ref[...], _nt_dims,
                        preferred_element_type=jnp.float32)
        + lax.dot_general(h_ref[...], whh_ref[...], _nt_dims,
                          preferred_element_type=jnp.float32)
        + bih_ref[...] + bhh_ref[...]
    )
    i_g = jax.nn.sigmoid(gates[:, 0:H])
    f_g = jax.nn.sigmoid(gates[:, H:2 * H])
    g_g = jnp.tanh(gates[:, 2 * H:3 * H])
    o_g = jax.nn.sigmoid(gates[:, 3 * H:4 * H])
    c_new = f_g * c_ref[...] + i_g * g_g
    cout_ref[...] = c_new
    hout_ref[...] = o_g * jnp.tanh(c_new)


def _lstm(x, h, c, W_ih, W_hh, b_ih2, b_hh2):
    return pl.pallas_call(
        _lstm_body,
        out_shape=[
            jax.ShapeDtypeStruct((B, H), jnp.float32),
            jax.ShapeDtypeStruct((B, H), jnp.float32),
        ],
    )(x, h, c, W_ih, W_hh, b_ih2, b_hh2)


def _fc_body(h_ref, fcb_ref, fcw_hbm, pred_ref, bufs, sems):
    i = pl.program_id(0)

    def fire(nxt):
        @pl.when(nxt < _NT)
        def _():
            slot = lax.rem(nxt, _NBUF)
            pltpu.make_async_copy(
                fcw_hbm.at[pl.ds(nxt * _SUB, _SUB)],
                bufs.at[slot], sems.at[slot]).start()

        @pl.when(nxt == _NT)
        def _():
            pltpu.make_async_copy(
                fcw_hbm.at[pl.ds(_NT * _SUB, _TAIL)],
                bufs.at[_NT % _NBUF, pl.ds(0, _TAIL)],
                sems.at[_NT % _NBUF]).start()

    @pl.when(i == 0)
    def _():
        for k in range(_NBUF - 1):
            fire(jnp.int32(k))

    fire(i + _NBUF - 1)

    @pl.when(i < _NT)
    def _():
        slot = lax.rem(i, _NBUF)
        pltpu.make_async_copy(
            fcw_hbm.at[pl.ds(i * _SUB, _SUB)],
            bufs.at[slot], sems.at[slot]).wait()
        pred_ref[...] = (
            lax.dot_general(h_ref[...], bufs[slot], _nt_dims,
                            preferred_element_type=jnp.float32)
            + fcb_ref[...]
        )

    @pl.when(i == _NT)
    def _():
        pltpu.make_async_copy(
            fcw_hbm.at[pl.ds(_NT * _SUB, _TAIL)],
            bufs.at[_NT % _NBUF, pl.ds(0, _TAIL)],
            sems.at[_NT % _NBUF]).wait()
        pred_ref[:, 0:_TAIL] = (
            lax.dot_general(h_ref[...], bufs[_NT % _NBUF, 0:_TAIL],
                            _nt_dims, preferred_element_type=jnp.float32)
            + fcb_ref[:, 0:_TAIL]
        )


def _fc(h_new, fc_W, fc_b2):
    return pl.pallas_call(
        _fc_body,
        grid=(_NV,),
        in_specs=[
            pl.BlockSpec((B, H), lambda i: (0, 0)),
            pl.BlockSpec((1, _SUB), lambda i: (0, i)),
            pl.BlockSpec(memory_space=pltpu.MemorySpace.HBM),
        ],
        out_specs=pl.BlockSpec((B, _SUB), lambda i: (0, i)),
        out_shape=jax.ShapeDtypeStruct((B, V), jnp.float32),
        scratch_shapes=[
            pltpu.VMEM((_NBUF, _SUB, H), jnp.float32),
            pltpu.SemaphoreType.DMA((_NBUF,)),
        ],
        compiler_params=pltpu.CompilerParams(
            vmem_limit_bytes=60 * 1024 * 1024),
    )(h_new, fc_b2, fc_W)


def kernel(input, h0, c0, emb, W_ih, W_hh, b_ih, b_hh, fc_W, fc_b):
    ids = input.astype(jnp.int32)
    x = _sc_gather(ids, emb)
    h_new, c_new = _lstm(x, h0[0], c0[0], W_ih, W_hh,
                         b_ih.reshape(1, 4 * H), b_hh.reshape(1, 4 * H))
    pred = _fc(h_new, fc_W, fc_b.reshape(1, V))
    return (pred, h_new[None, :, :], c_new[None, :, :])
